# baseline (speedup 1.0000x reference)
import functools

import jax
import jax.numpy as jnp
from jax import lax
from jax.experimental import pallas as pl
from jax.experimental.pallas import tpu as pltpu
from jax.experimental.pallas import tpu_sc as plsc

_N = 10000
_E = 160000
_R = 8
_D = 256
_L = 16
_NSEG = _N * _R
_NSEG_PAD = 80384
_EPT = _E // 16
_NB = 10
_BB = _EPT // _NB
_ROWS_PT = _NSEG // 16

_mesh = plsc.VectorSubcoreMesh(core_axis_name="c", subcore_axis_name="s")


def _seg_cnt_body(dst_hbm, typ_hbm, seg_hbm, inv_hbm,
                  dst_v, typ_v, seg_v, ones_v, zero_v, tmp_v, cnt_sp):
    c = lax.axis_index("c")
    s = lax.axis_index("s")
    base = s * _EPT
    pltpu.sync_copy(dst_hbm.at[pl.ds(base, _EPT)], dst_v)
    pltpu.sync_copy(typ_hbm.at[pl.ds(base, _EPT)], typ_v)

    def seg_loop(i, carry):
        sl = pl.ds(i * _L, _L)
        seg_v[sl] = dst_v[sl] * _R + typ_v[sl]
        ones_v[sl] = jnp.full((_L,), 1.0, jnp.float32)
        return carry
    lax.fori_loop(0, _EPT // _L, seg_loop, 0)

    zpt = _NSEG_PAD // 16

    def zero_loop(i, carry):
        zero_v[pl.ds(i * _L, _L)] = jnp.zeros((_L,), jnp.float32)
        return carry
    lax.fori_loop(0, zpt // _L, zero_loop, 0)
    pltpu.sync_copy(zero_v, cnt_sp.at[pl.ds(s * zpt, zpt)])
    plsc.subcore_barrier()

    pltpu.sync_copy(ones_v, cnt_sp.at[seg_v], add=True)
    plsc.subcore_barrier()

    ipt = _NSEG_PAD // 32
    off = c * (_NSEG_PAD // 2) + s * ipt
    pltpu.sync_copy(cnt_sp.at[pl.ds(off, ipt)], tmp_v)

    def inv_loop(i, carry):
        sl = pl.ds(i * _L, _L)
        tmp_v[sl] = 1.0 / jnp.maximum(tmp_v[sl], 1.0)
        return carry
    lax.fori_loop(0, ipt // _L, inv_loop, 0)
    pltpu.sync_copy(tmp_v, inv_hbm.at[pl.ds(off, ipt)])

    @pl.when(c == 0)
    def _():
        pltpu.sync_copy(seg_v, seg_hbm.at[pl.ds(base, _EPT)])


_seg_cnt_call = pl.kernel(
    _seg_cnt_body,
    out_type=[
        jax.ShapeDtypeStruct((_E,), jnp.int32),
        jax.ShapeDtypeStruct((_NSEG_PAD,), jnp.float32),
    ],
    mesh=_mesh,
    compiler_params=pltpu.CompilerParams(use_tc_tiling_on_sc=False),
    scratch_types=[
        pltpu.VMEM((_EPT,), jnp.int32),
        pltpu.VMEM((_EPT,), jnp.int32),
        pltpu.VMEM((_EPT,), jnp.int32),
        pltpu.VMEM((_EPT,), jnp.float32),
        pltpu.VMEM((_NSEG_PAD // 16,), jnp.float32),
        pltpu.VMEM((_NSEG_PAD // 32,), jnp.float32),
        pltpu.VMEM_SHARED((_NSEG_PAD,), jnp.float32),
    ],
)


def _scatter_body(xt_hbm, src_hbm, seg_hbm, asum_hbm,
                  src_a, src_b, seg_a, seg_b, rows_a, rows_b, acc_sp,
                  sem_a, sem_b):
    c = lax.axis_index("c")
    s = lax.axis_index("s")
    srcs = (src_a, src_b)
    segs = (seg_a, seg_b)
    rows = (rows_a, rows_b)
    sems = (sem_a, sem_b)

    def chunk_loop(cc, carry):
        chunk = c * 8 + cc
        row0 = s * _ROWS_PT

        def zbuf_loop(i, carry2):
            rows_a[i, :] = jnp.zeros((_L,), jnp.float32)
            return carry2
        lax.fori_loop(0, _BB, zbuf_loop, 0)

        def zcp_loop(i, carry2):
            pltpu.sync_copy(rows_a, acc_sp.at[pl.ds(row0 + i * _BB, _BB), :])
            return carry2
        lax.fori_loop(0, _ROWS_PT // _BB, zcp_loop, 0)
        plsc.subcore_barrier()

        pltpu.sync_copy(src_hbm.at[s, 0], src_a)
        pltpu.sync_copy(seg_hbm.at[s, 0], seg_a)
        pending = pltpu.async_copy(xt_hbm.at[chunk].at[src_a], rows_a, sem_a)
        for b in range(_NB):
            p = b % 2
            q = (b + 1) % 2
            if b + 1 < _NB:
                pltpu.sync_copy(src_hbm.at[s, b + 1], srcs[q])
                pltpu.sync_copy(seg_hbm.at[s, b + 1], segs[q])
            pending.wait()
            if b + 1 < _NB:
                pending = pltpu.async_copy(
                    xt_hbm.at[chunk].at[srcs[q]], rows[q], sems[q])
            pltpu.sync_copy(rows[p], acc_sp.at[segs[p]], add=True)
        plsc.subcore_barrier()

        pltpu.sync_copy(
            acc_sp.at[pl.ds(row0, _ROWS_PT), :],
            asum_hbm.at[chunk, pl.ds(row0, _ROWS_PT), :])
        return carry
    lax.fori_loop(0, 8, chunk_loop, 0)


_scatter_call = pl.kernel(
    _scatter_body,
    out_type=jax.ShapeDtypeStruct((_L, _NSEG, _L), jnp.float32),
    mesh=_mesh,
    compiler_params=pltpu.CompilerParams(use_tc_tiling_on_sc=False),
    scratch_types=[
        pltpu.VMEM((_BB,), jnp.int32),
        pltpu.VMEM((_BB,), jnp.int32),
        pltpu.VMEM((_BB,), jnp.int32),
        pltpu.VMEM((_BB,), jnp.int32),
        pltpu.VMEM((_BB, _L), jnp.float32),
        pltpu.VMEM((_BB, _L), jnp.float32),
        pltpu.VMEM_SHARED((_NSEG, _L), jnp.float32),
        pltpu.SemaphoreType.DMA,
        pltpu.SemaphoreType.DMA,
    ],
)


def _dense_body(relu, acm_ref, inv_ref, x_ref, wp_ref, root_ref, b_ref,
                out_ref):
    acc = jnp.dot(x_ref[...], root_ref[...],
                  preferred_element_type=jnp.float32)
    for c in range(_L):
        scaled = acm_ref[c] * inv_ref[...]
        acc += jnp.dot(scaled, wp_ref[c], preferred_element_type=jnp.float32)
    acc += b_ref[...]
    if relu:
        acc = jnp.maximum(acc, 0.0)
    out_ref[...] = acc


def _dense_call(acm, inv_exp, x, wp, root, b2d, relu):
    bn = 1000
    grid = (_N // bn,)
    return pl.pallas_call(
        functools.partial(_dense_body, relu),
        grid=grid,
        in_specs=[
            pl.BlockSpec((_L, bn, _R * _L), lambda i: (0, i, 0)),
            pl.BlockSpec((bn, _R * _L), lambda i: (i, 0)),
            pl.BlockSpec((bn, _D), lambda i: (i, 0)),
            pl.BlockSpec((_L, _R * _L, _D), lambda i: (0, 0, 0)),
            pl.BlockSpec((_D, _D), lambda i: (0, 0)),
            pl.BlockSpec((1, _D), lambda i: (0, 0)),
        ],
        out_specs=pl.BlockSpec((bn, _D), lambda i: (i, 0)),
        out_shape=jax.ShapeDtypeStruct((_N, _D), jnp.float32),
        compiler_params=pltpu.CompilerParams(
            dimension_semantics=("parallel",)),
    )(acm, inv_exp, x, wp, root, b2d)


def _chunk_major(h):
    return h.reshape(_N, _L, _L).transpose(1, 0, 2)


def kernel(x, edge_index, edge_type, W1, root1, b1, W2, root2, b2):
    src = edge_index[0]
    dst = edge_index[1]
    seg, inv_pad = _seg_cnt_call(dst, edge_type.astype(jnp.int32))
    inv_exp = jnp.broadcast_to(
        inv_pad[:_NSEG].reshape(_N, _R, 1), (_N, _R, _L)).reshape(_N, _R * _L)
    src2 = src.reshape(16, _NB, _BB)
    seg2 = seg.reshape(16, _NB, _BB)

    def _wperm(w):
        return w.reshape(_R, _L, _L, _D).transpose(1, 0, 2, 3).reshape(
            _L, _R * _L, _D)

    asum1 = _scatter_call(_chunk_major(x), src2, seg2)
    h = _dense_call(asum1.reshape(_L, _N, _R * _L), inv_exp, x, _wperm(W1),
                    root1, b1.reshape(1, _D), relu=True)

    asum2 = _scatter_call(_chunk_major(h), src2, seg2)
    out = _dense_call(asum2.reshape(_L, _N, _R * _L), inv_exp, h, _wperm(W2),
                      root2, b2.reshape(1, _D), relu=False)
    return out


# speedup vs baseline: 1.1602x; 1.1602x over previous
import functools

import jax
import jax.numpy as jnp
from jax import lax
from jax.experimental import pallas as pl
from jax.experimental.pallas import tpu as pltpu
from jax.experimental.pallas import tpu_sc as plsc

_N = 10000
_E = 160000
_R = 8
_D = 256
_L = 16
_NSEG = _N * _R
_NSEG_PAD = 80384
_EPT = _E // 16
_NB = 10
_BB = _EPT // _NB
_ROWS_PT = _NSEG // 16

_mesh = plsc.VectorSubcoreMesh(core_axis_name="c", subcore_axis_name="s")


def _seg_cnt_body(src_hbm, dst_hbm, typ_hbm, seg_hbm, inv_hbm, src8_hbm,
                  src_v, s8_v, dst_v, typ_v, seg_v, ones_v, zero_v, tmp_v,
                  cnt_sp):
    c = lax.axis_index("c")
    s = lax.axis_index("s")
    base = s * _EPT
    pltpu.sync_copy(src_hbm.at[pl.ds(base, _EPT)], src_v)
    pltpu.sync_copy(dst_hbm.at[pl.ds(base, _EPT)], dst_v)
    pltpu.sync_copy(typ_hbm.at[pl.ds(base, _EPT)], typ_v)

    @pl.when(c == 0)
    def _():
        def cg_loop(cg, carry):
            def sh_loop(i, carry2):
                sl = pl.ds(i * _L, _L)
                s8_v[sl] = src_v[sl] * 8 + cg
                return carry2
            lax.fori_loop(0, _EPT // _L, sh_loop, 0)
            pltpu.sync_copy(s8_v, src8_hbm.at[cg, s])
            return carry
        lax.fori_loop(0, 8, cg_loop, 0)

    def seg_loop(i, carry):
        sl = pl.ds(i * _L, _L)
        seg_v[sl] = dst_v[sl] * _R + typ_v[sl]
        ones_v[sl] = jnp.full((_L,), 1.0, jnp.float32)
        return carry
    lax.fori_loop(0, _EPT // _L, seg_loop, 0)

    zpt = _NSEG_PAD // 16

    def zero_loop(i, carry):
        zero_v[pl.ds(i * _L, _L)] = jnp.zeros((_L,), jnp.float32)
        return carry
    lax.fori_loop(0, zpt // _L, zero_loop, 0)
    pltpu.sync_copy(zero_v, cnt_sp.at[pl.ds(s * zpt, zpt)])
    plsc.subcore_barrier()

    pltpu.sync_copy(ones_v, cnt_sp.at[seg_v], add=True)
    plsc.subcore_barrier()

    ipt = _NSEG_PAD // 32
    off = c * (_NSEG_PAD // 2) + s * ipt
    pltpu.sync_copy(cnt_sp.at[pl.ds(off, ipt)], tmp_v)

    def inv_loop(i, carry):
        sl = pl.ds(i * _L, _L)
        tmp_v[sl] = 1.0 / jnp.maximum(tmp_v[sl], 1.0)
        return carry
    lax.fori_loop(0, ipt // _L, inv_loop, 0)
    pltpu.sync_copy(tmp_v, inv_hbm.at[pl.ds(off, ipt)])

    @pl.when(c == 0)
    def _():
        pltpu.sync_copy(seg_v, seg_hbm.at[pl.ds(base, _EPT)])


_seg_cnt_call = pl.kernel(
    _seg_cnt_body,
    out_type=[
        jax.ShapeDtypeStruct((_E,), jnp.int32),
        jax.ShapeDtypeStruct((_NSEG_PAD,), jnp.float32),
        jax.ShapeDtypeStruct((8, 16, _EPT), jnp.int32),
    ],
    mesh=_mesh,
    compiler_params=pltpu.CompilerParams(use_tc_tiling_on_sc=False),
    scratch_types=[
        pltpu.VMEM((_EPT,), jnp.int32),
        pltpu.VMEM((_EPT,), jnp.int32),
        pltpu.VMEM((_EPT,), jnp.int32),
        pltpu.VMEM((_EPT,), jnp.int32),
        pltpu.VMEM((_EPT,), jnp.int32),
        pltpu.VMEM((_EPT,), jnp.float32),
        pltpu.VMEM((_NSEG_PAD // 16,), jnp.float32),
        pltpu.VMEM((_NSEG_PAD // 32,), jnp.float32),
        pltpu.VMEM_SHARED((_NSEG_PAD,), jnp.float32),
    ],
)


def _scatter_body(xt_hbm, src_hbm, seg_hbm, asum_hbm,
                  src_a, src_b, seg_a, seg_b, rows_a, rows_b, acc_sp,
                  sem_a, sem_b):
    c = lax.axis_index("c")
    s = lax.axis_index("s")
    srcs = (src_a, src_b)
    segs = (seg_a, seg_b)
    rows = (rows_a, rows_b)
    sems = (sem_a, sem_b)

    def chunk_loop(cc, carry):
        chunk = c * 8 + cc
        row0 = s * _ROWS_PT

        def zbuf_loop(i, carry2):
            rows_a[i, :] = jnp.zeros((_L,), jnp.float32)
            return carry2
        lax.fori_loop(0, _BB, zbuf_loop, 0)

        def zcp_loop(i, carry2):
            pltpu.sync_copy(rows_a, acc_sp.at[pl.ds(row0 + i * _BB, _BB), :])
            return carry2
        lax.fori_loop(0, _ROWS_PT // _BB, zcp_loop, 0)
        plsc.subcore_barrier()

        pltpu.sync_copy(src_hbm.at[cc, s, 0], src_a)
        pltpu.sync_copy(seg_hbm.at[s, 0], seg_a)
        pending = pltpu.async_copy(xt_hbm.at[c].at[src_a], rows_a, sem_a)
        for b in range(_NB):
            p = b % 2
            q = (b + 1) % 2
            if b + 1 < _NB:
                pltpu.sync_copy(src_hbm.at[cc, s, b + 1], srcs[q])
                pltpu.sync_copy(seg_hbm.at[s, b + 1], segs[q])
            pending.wait()
            if b + 1 < _NB:
                pending = pltpu.async_copy(
                    xt_hbm.at[c].at[srcs[q]], rows[q], sems[q])
            pltpu.sync_copy(rows[p], acc_sp.at[segs[p]], add=True)
        plsc.subcore_barrier()

        pltpu.sync_copy(
            acc_sp.at[pl.ds(row0, _ROWS_PT), :],
            asum_hbm.at[chunk, pl.ds(row0, _ROWS_PT), :])
        return carry
    lax.fori_loop(0, 8, chunk_loop, 0)


_scatter_call = pl.kernel(
    _scatter_body,
    out_type=jax.ShapeDtypeStruct((_L, _NSEG, _L), jnp.float32),
    mesh=_mesh,
    compiler_params=pltpu.CompilerParams(use_tc_tiling_on_sc=False),
    scratch_types=[
        pltpu.VMEM((_BB,), jnp.int32),
        pltpu.VMEM((_BB,), jnp.int32),
        pltpu.VMEM((_BB,), jnp.int32),
        pltpu.VMEM((_BB,), jnp.int32),
        pltpu.VMEM((_BB, _L), jnp.float32),
        pltpu.VMEM((_BB, _L), jnp.float32),
        pltpu.VMEM_SHARED((_NSEG, _L), jnp.float32),
        pltpu.SemaphoreType.DMA,
        pltpu.SemaphoreType.DMA,
    ],
)


def _dense_body(relu, split_out, acm_ref, inv_ref, x_ref, wp_ref, root_ref,
                b_ref, out_ref):
    xb = jnp.concatenate([x_ref[0], x_ref[1]], axis=-1)
    acc = jnp.dot(xb, root_ref[...], preferred_element_type=jnp.float32)
    for c in range(_L):
        scaled = acm_ref[c] * inv_ref[...]
        acc += jnp.dot(scaled, wp_ref[c], preferred_element_type=jnp.float32)
    acc += b_ref[...]
    if relu:
        acc = jnp.maximum(acc, 0.0)
    if split_out:
        out_ref[0] = acc[:, :_D // 2]
        out_ref[1] = acc[:, _D // 2:]
    else:
        out_ref[...] = acc


def _dense_call(acm, inv_exp, x2, wp, root, b2d, relu, split_out):
    bn = 1000
    grid = (_N // bn,)
    if split_out:
        out_spec = pl.BlockSpec((2, bn, _D // 2), lambda i: (0, i, 0))
        out_shape = jax.ShapeDtypeStruct((2, _N, _D // 2), jnp.float32)
    else:
        out_spec = pl.BlockSpec((bn, _D), lambda i: (i, 0))
        out_shape = jax.ShapeDtypeStruct((_N, _D), jnp.float32)
    return pl.pallas_call(
        functools.partial(_dense_body, relu, split_out),
        grid=grid,
        in_specs=[
            pl.BlockSpec((_L, bn, _R * _L), lambda i: (0, i, 0)),
            pl.BlockSpec((bn, _R * _L), lambda i: (i, 0)),
            pl.BlockSpec((2, bn, _D // 2), lambda i: (0, i, 0)),
            pl.BlockSpec((_L, _R * _L, _D), lambda i: (0, 0, 0)),
            pl.BlockSpec((_D, _D), lambda i: (0, 0)),
            pl.BlockSpec((1, _D), lambda i: (0, 0)),
        ],
        out_specs=out_spec,
        out_shape=out_shape,
        compiler_params=pltpu.CompilerParams(
            dimension_semantics=("parallel",)),
    )(acm, inv_exp, x2, wp, root, b2d)


def kernel(x, edge_index, edge_type, W1, root1, b1, W2, root2, b2):
    src = edge_index[0]
    dst = edge_index[1]
    seg, inv_pad, src8 = _seg_cnt_call(src, dst, edge_type.astype(jnp.int32))
    inv_exp = jnp.broadcast_to(
        inv_pad[:_NSEG].reshape(_N, _R, 1), (_N, _R, _L)).reshape(_N, _R * _L)
    src8r = src8.reshape(8, 16, _NB, _BB)
    seg2 = seg.reshape(16, _NB, _BB)

    def _wperm(w):
        return w.reshape(_R, _L, _L, _D).transpose(1, 0, 2, 3).reshape(
            _L, _R * _L, _D)

    x2 = jnp.swapaxes(x.reshape(_N, 2, _D // 2), 0, 1)

    asum1 = _scatter_call(x2.reshape(2, _N * 8, _L), src8r, seg2)
    h2 = _dense_call(asum1.reshape(_L, _N, _R * _L), inv_exp, x2, _wperm(W1),
                     root1, b1.reshape(1, _D), relu=True, split_out=True)

    asum2 = _scatter_call(h2.reshape(2, _N * 8, _L), src8r, seg2)
    out = _dense_call(asum2.reshape(_L, _N, _R * _L), inv_exp, h2, _wperm(W2),
                      root2, b2.reshape(1, _D), relu=False, split_out=False)
    return out


# speedup vs baseline: 1.1785x; 1.0158x over previous
import functools

import jax
import jax.numpy as jnp
from jax import lax
from jax.experimental import pallas as pl
from jax.experimental.pallas import tpu as pltpu
from jax.experimental.pallas import tpu_sc as plsc

_N = 10000
_E = 160000
_R = 8
_D = 256
_L = 16
_NSEG = _N * _R
_NSEG_PAD = 80384
_EPT = _E // 16
_NB = 10
_BB = _EPT // _NB
_ROWS_PT = _NSEG // 16

_mesh = plsc.VectorSubcoreMesh(core_axis_name="c", subcore_axis_name="s")


def _seg_cnt_body(src_hbm, dst_hbm, typ_hbm, seg_hbm, inv_hbm, src8_hbm,
                  src_v, s8_v, dst_v, typ_v, seg_v, ones_v, zero_v, tmp_v,
                  cnt_sp):
    c = lax.axis_index("c")
    s = lax.axis_index("s")
    base = s * _EPT
    pltpu.sync_copy(src_hbm.at[pl.ds(base, _EPT)], src_v)
    pltpu.sync_copy(dst_hbm.at[pl.ds(base, _EPT)], dst_v)
    pltpu.sync_copy(typ_hbm.at[pl.ds(base, _EPT)], typ_v)

    @pl.when(c == 0)
    def _():
        def cg_loop(cg, carry):
            def sh_loop(i, carry2):
                sl = pl.ds(i * _L, _L)
                s8_v[sl] = src_v[sl] * 8 + cg
                return carry2
            lax.fori_loop(0, _EPT // _L, sh_loop, 0)
            pltpu.sync_copy(s8_v, src8_hbm.at[cg, s])
            return carry
        lax.fori_loop(0, 8, cg_loop, 0)

    def seg_loop(i, carry):
        sl = pl.ds(i * _L, _L)
        seg_v[sl] = dst_v[sl] * _R + typ_v[sl]
        ones_v[sl] = jnp.full((_L,), 1.0, jnp.float32)
        return carry
    lax.fori_loop(0, _EPT // _L, seg_loop, 0)

    zpt = _NSEG_PAD // 16

    def zero_loop(i, carry):
        zero_v[pl.ds(i * _L, _L)] = jnp.zeros((_L,), jnp.float32)
        return carry
    lax.fori_loop(0, zpt // _L, zero_loop, 0)
    pltpu.sync_copy(zero_v, cnt_sp.at[pl.ds(s * zpt, zpt)])
    plsc.subcore_barrier()

    pltpu.sync_copy(ones_v, cnt_sp.at[seg_v], add=True)
    plsc.subcore_barrier()

    ipt = _NSEG_PAD // 32
    off = c * (_NSEG_PAD // 2) + s * ipt
    pltpu.sync_copy(cnt_sp.at[pl.ds(off, ipt)], tmp_v)

    def inv_loop(i, carry):
        sl = pl.ds(i * _L, _L)
        tmp_v[sl] = 1.0 / jnp.maximum(tmp_v[sl], 1.0)
        return carry
    lax.fori_loop(0, ipt // _L, inv_loop, 0)
    pltpu.sync_copy(tmp_v, inv_hbm.at[pl.ds(off, ipt)])

    @pl.when(c == 0)
    def _():
        pltpu.sync_copy(seg_v, seg_hbm.at[pl.ds(base, _EPT)])


_seg_cnt_call = pl.kernel(
    _seg_cnt_body,
    out_type=[
        jax.ShapeDtypeStruct((_E,), jnp.int32),
        jax.ShapeDtypeStruct((_NSEG_PAD,), jnp.float32),
        jax.ShapeDtypeStruct((8, 16, _EPT), jnp.int32),
    ],
    mesh=_mesh,
    compiler_params=pltpu.CompilerParams(use_tc_tiling_on_sc=False),
    scratch_types=[
        pltpu.VMEM((_EPT,), jnp.int32),
        pltpu.VMEM((_EPT,), jnp.int32),
        pltpu.VMEM((_EPT,), jnp.int32),
        pltpu.VMEM((_EPT,), jnp.int32),
        pltpu.VMEM((_EPT,), jnp.int32),
        pltpu.VMEM((_EPT,), jnp.float32),
        pltpu.VMEM((_NSEG_PAD // 16,), jnp.float32),
        pltpu.VMEM((_NSEG_PAD // 32,), jnp.float32),
        pltpu.VMEM_SHARED((_NSEG_PAD,), jnp.float32),
    ],
)


def _scatter_body(xt_hbm, src_hbm, seg_hbm, asum_hbm,
                  src_a, src_b, seg_a, seg_b, rows_a, rows_b, acc_sp,
                  sem_a, sem_b):
    c = lax.axis_index("c")
    s = lax.axis_index("s")
    srcs = (src_a, src_b)
    segs = (seg_a, seg_b)
    rows = (rows_a, rows_b)
    sems = (sem_a, sem_b)

    def chunk_loop(cc, carry):
        chunk = c * 8 + cc
        row0 = s * _ROWS_PT

        pltpu.sync_copy(src_hbm.at[cc, s, 0], src_a)
        pltpu.sync_copy(seg_hbm.at[s, 0], seg_a)
        pending = pltpu.async_copy(xt_hbm.at[c].at[src_a], rows_b, sem_b)

        def zbuf_loop(i, carry2):
            rows_a[i, :] = jnp.zeros((_L,), jnp.float32)
            return carry2
        lax.fori_loop(0, _BB, zbuf_loop, 0)

        def zcp_loop(i, carry2):
            pltpu.sync_copy(rows_a, acc_sp.at[pl.ds(row0 + i * _BB, _BB), :])
            return carry2
        lax.fori_loop(0, _ROWS_PT // _BB, zcp_loop, 0)
        plsc.subcore_barrier()

        for b in range(_NB):
            p = b % 2
            q = (b + 1) % 2
            if b + 1 < _NB:
                pltpu.sync_copy(src_hbm.at[cc, s, b + 1], srcs[q])
                pltpu.sync_copy(seg_hbm.at[s, b + 1], segs[q])
            pending.wait()
            if b + 1 < _NB:
                pending = pltpu.async_copy(
                    xt_hbm.at[c].at[srcs[q]], rows[p], sems[p])
            pltpu.sync_copy(rows[q], acc_sp.at[segs[p]], add=True)
        plsc.subcore_barrier()

        pltpu.sync_copy(
            acc_sp.at[pl.ds(row0, _ROWS_PT), :],
            asum_hbm.at[chunk, pl.ds(row0, _ROWS_PT), :])
        return carry
    lax.fori_loop(0, 8, chunk_loop, 0)


_scatter_call = pl.kernel(
    _scatter_body,
    out_type=jax.ShapeDtypeStruct((_L, _NSEG, _L), jnp.float32),
    mesh=_mesh,
    compiler_params=pltpu.CompilerParams(use_tc_tiling_on_sc=False),
    scratch_types=[
        pltpu.VMEM((_BB,), jnp.int32),
        pltpu.VMEM((_BB,), jnp.int32),
        pltpu.VMEM((_BB,), jnp.int32),
        pltpu.VMEM((_BB,), jnp.int32),
        pltpu.VMEM((_BB, _L), jnp.float32),
        pltpu.VMEM((_BB, _L), jnp.float32),
        pltpu.VMEM_SHARED((_NSEG, _L), jnp.float32),
        pltpu.SemaphoreType.DMA,
        pltpu.SemaphoreType.DMA,
    ],
)


def _dense_body(relu, split_out, acm_ref, inv_ref, x_ref, wp_ref, root_ref,
                b_ref, out_ref):
    xb = jnp.concatenate([x_ref[0], x_ref[1]], axis=-1)
    acc = jnp.dot(xb, root_ref[...], preferred_element_type=jnp.float32)
    for c in range(_L):
        scaled = acm_ref[c] * inv_ref[...]
        acc += jnp.dot(scaled, wp_ref[c], preferred_element_type=jnp.float32)
    acc += b_ref[...]
    if relu:
        acc = jnp.maximum(acc, 0.0)
    if split_out:
        out_ref[0] = acc[:, :_D // 2]
        out_ref[1] = acc[:, _D // 2:]
    else:
        out_ref[...] = acc


def _dense_call(acm, inv_exp, x2, wp, root, b2d, relu, split_out):
    bn = 1000
    grid = (_N // bn,)
    if split_out:
        out_spec = pl.BlockSpec((2, bn, _D // 2), lambda i: (0, i, 0))
        out_shape = jax.ShapeDtypeStruct((2, _N, _D // 2), jnp.float32)
    else:
        out_spec = pl.BlockSpec((bn, _D), lambda i: (i, 0))
        out_shape = jax.ShapeDtypeStruct((_N, _D), jnp.float32)
    return pl.pallas_call(
        functools.partial(_dense_body, relu, split_out),
        grid=grid,
        in_specs=[
            pl.BlockSpec((_L, bn, _R * _L), lambda i: (0, i, 0)),
            pl.BlockSpec((bn, _R * _L), lambda i: (i, 0)),
            pl.BlockSpec((2, bn, _D // 2), lambda i: (0, i, 0)),
            pl.BlockSpec((_L, _R * _L, _D), lambda i: (0, 0, 0)),
            pl.BlockSpec((_D, _D), lambda i: (0, 0)),
            pl.BlockSpec((1, _D), lambda i: (0, 0)),
        ],
        out_specs=out_spec,
        out_shape=out_shape,
        compiler_params=pltpu.CompilerParams(
            dimension_semantics=("parallel",)),
    )(acm, inv_exp, x2, wp, root, b2d)


def kernel(x, edge_index, edge_type, W1, root1, b1, W2, root2, b2):
    src = edge_index[0]
    dst = edge_index[1]
    seg, inv_pad, src8 = _seg_cnt_call(src, dst, edge_type.astype(jnp.int32))
    inv_exp = jnp.broadcast_to(
        inv_pad[:_NSEG].reshape(_N, _R, 1), (_N, _R, _L)).reshape(_N, _R * _L)
    src8r = src8.reshape(8, 16, _NB, _BB)
    seg2 = seg.reshape(16, _NB, _BB)

    def _wperm(w):
        return w.reshape(_R, _L, _L, _D).transpose(1, 0, 2, 3).reshape(
            _L, _R * _L, _D)

    x2 = jnp.swapaxes(x.reshape(_N, 2, _D // 2), 0, 1)

    asum1 = _scatter_call(x2.reshape(2, _N * 8, _L), src8r, seg2)
    h2 = _dense_call(asum1.reshape(_L, _N, _R * _L), inv_exp, x2, _wperm(W1),
                     root1, b1.reshape(1, _D), relu=True, split_out=True)

    asum2 = _scatter_call(h2.reshape(2, _N * 8, _L), src8r, seg2)
    out = _dense_call(asum2.reshape(_L, _N, _R * _L), inv_exp, h2, _wperm(W2),
                      root2, b2.reshape(1, _D), relu=False, split_out=False)
    return out
